# Initial kernel scaffold; baseline (speedup 1.0000x reference)
import jax
import jax.numpy as jnp
from jax.experimental import pallas as pl


def kernel(x, edge_index, edge_attr, batch, W1, b1, W2, b2, W3, b3, Wc, bc):
    raise NotImplementedError("write your pallas kernel here")


# speedup vs baseline: 12.4369x; 12.4369x over previous
import functools

import jax
import jax.numpy as jnp
from jax import lax
from jax.experimental import pallas as pl
from jax.experimental.pallas import tpu as pltpu
from jax.experimental.pallas import tpu_sc as plsc

N = 10000
E = 320000
D_IN = 128
H = 64
C = 100
B = 16

NC = 2
NS = 16
NW = NC * NS
CHUNK = 128
ROWS_A = 632
ROWS_B = N - (NS - 1) * ROWS_A

CHUNKS_PER_TILE = -(-E // (NW * CHUNK))
E_PAD = NW * CHUNK * CHUNKS_PER_TILE
E_PER_TILE = E_PAD // NW
EH_PER_TILE = E // NW

_MESH = plsc.VectorSubcoreMesh(core_axis_name="c", subcore_axis_name="s",
                               num_cores=NC, num_subcores=NS)


def _wid():
  return lax.axis_index("s") * NC + lax.axis_index("c")


def _per_tile_copy(sid, make_src, make_dst):
  off = pl.multiple_of(sid * ROWS_A, 8)

  @pl.when(sid < NS - 1)
  def _():
    pltpu.sync_copy(make_src(off, ROWS_A), make_dst(off, ROWS_A))

  @pl.when(sid == NS - 1)
  def _():
    pltpu.sync_copy(make_src(off, ROWS_B), make_dst(off, ROWS_B))


@functools.partial(
    pl.kernel,
    out_type=jax.ShapeDtypeStruct((NC, N, 16), jnp.float32),
    mesh=_MESH,
    scratch_types=[
        pltpu.VMEM((CHUNK,), jnp.int32),
        pltpu.VMEM((CHUNK, 16), jnp.float32),
        pltpu.VMEM_SHARED((N + 16, 16), jnp.float32),
    ],
)
def _sc_hist(colh_hbm, zeros_hbm, out_hbm, col_v, ones_v, acc_sh):
  cid = lax.axis_index("c")
  sid = lax.axis_index("s")
  wid = sid * NC + cid
  lanes = lax.iota(jnp.int32, 16)
  e0 = jnp.where(lanes == 0, 1.0, 0.0).astype(jnp.float32)
  for i in range(CHUNK):
    ones_v[i] = e0
  _per_tile_copy(sid, lambda o, r: zeros_hbm.at[pl.ds(0, r)],
                 lambda o, r: acc_sh.at[pl.ds(o, r)])

  @pl.when(sid == 0)
  def _():
    pltpu.sync_copy(zeros_hbm.at[pl.ds(0, 16)], acc_sh.at[pl.ds(N, 16)])

  plsc.subcore_barrier()

  def chunk(j, _):
    base = wid * E_PER_TILE + j * CHUNK
    pltpu.sync_copy(colh_hbm.at[pl.ds(base, CHUNK)], col_v)
    pltpu.sync_copy(ones_v, acc_sh.at[col_v], add=True)
    return ()

  lax.fori_loop(0, CHUNKS_PER_TILE, chunk, ())
  plsc.subcore_barrier()
  _per_tile_copy(sid, lambda o, r: acc_sh.at[pl.ds(o, r)],
                 lambda o, r: out_hbm.at[cid, pl.ds(o, r)])


def _edge_pass_body(first, hw_hbm, row_hbm, col_hbm, w_hbm,
                    zeros_hbm, part_hbm, w_out_hbm,
                    row_v, col_v, w_v, msg_v, acc_sh, sem):
  cid = lax.axis_index("c")
  sid = lax.axis_index("s")
  wid = sid * NC + cid
  _per_tile_copy(sid, lambda o, r: zeros_hbm.at[pl.ds(0, r)],
                 lambda o, r: acc_sh.at[pl.ds(o, r)])
  plsc.subcore_barrier()

  def chunk(j, _):
    base = wid * E_PER_TILE + j * CHUNK
    pltpu.sync_copy(row_hbm.at[pl.ds(base, CHUNK)], row_v)
    gather = pltpu.async_copy(hw_hbm.at[row_v], msg_v, sem)
    pltpu.sync_copy(col_hbm.at[pl.ds(base, CHUNK)], col_v)
    pltpu.sync_copy(w_hbm.at[pl.ds(base, CHUNK)], w_v)
    if first:
      def wbody(k, _):
        sl = pl.ds(k * 16, 16)
        w_v[sl] = jnp.exp(-w_v[sl])
        return ()

      lax.fori_loop(0, CHUNK // 16, wbody, (), unroll=8)
      pltpu.sync_copy(w_v, w_out_hbm.at[pl.ds(base, CHUNK)])
    gather.wait()

    def scale(g, _):
      wv = w_v[pl.ds(g * 16, 16)]
      for l in range(16):
        s = jnp.full((16,), wv[l], jnp.float32)
        r_idx = g * 16 + l
        for r in range(H // 16):
          sl = pl.ds(r * 16, 16)
          msg_v[r_idx, sl] = msg_v[r_idx, sl] * s
      return ()

    lax.fori_loop(0, CHUNK // 16, scale, ())
    pltpu.sync_copy(msg_v, acc_sh.at[col_v], add=True)
    return ()

  lax.fori_loop(0, CHUNKS_PER_TILE, chunk, ())
  plsc.subcore_barrier()
  _per_tile_copy(sid, lambda o, r: acc_sh.at[pl.ds(o, r)],
                 lambda o, r: part_hbm.at[cid, pl.ds(o, r)])


def _make_edge_pass(first):
  scratch = [
      pltpu.VMEM((CHUNK,), jnp.int32),
      pltpu.VMEM((CHUNK,), jnp.int32),
      pltpu.VMEM((CHUNK,), jnp.float32),
      pltpu.VMEM((CHUNK, H), jnp.float32),
      pltpu.VMEM_SHARED((N, H), jnp.float32),
      pltpu.SemaphoreType.DMA,
  ]
  part_t = jax.ShapeDtypeStruct((NC, N, H), jnp.float32)

  if first:
    out_type = (part_t, jax.ShapeDtypeStruct((E_PAD,), jnp.float32))

    def body(hw, row, col, ea, zeros, part, w_out, *scr):
      _edge_pass_body(True, hw, row, col, ea, zeros, part, w_out, *scr)
  else:
    out_type = part_t

    def body(hw, row, col, w, zeros, part, *scr):
      _edge_pass_body(False, hw, row, col, w, zeros, part, None, *scr)

  return pl.kernel(body, out_type=out_type, mesh=_MESH,
                   scratch_types=scratch,
                   compiler_params=pltpu.CompilerParams(
                       use_tc_tiling_on_sc=False))


_edge_pass_first = _make_edge_pass(True)
_edge_pass_rest = _make_edge_pass(False)


def _dot_t(a, w):
  return lax.dot_general(a, w, (((1,), (1,)), ((), ())),
                         preferred_element_type=jnp.float32)


def _tc_prep_body(hist_ref, x_ref, w1_ref, dis_ref, hws_ref):
  deg = hist_ref[0, :, 0:1] + hist_ref[1, :, 0:1] + 1.0
  dis = lax.rsqrt(deg)
  dis_ref[...] = dis
  hws_ref[...] = dis * _dot_t(x_ref[...], w1_ref[...])


_tc_prep = pl.pallas_call(
    _tc_prep_body,
    out_shape=(
        jax.ShapeDtypeStruct((N, 1), jnp.float32),
        jax.ShapeDtypeStruct((N, H), jnp.float32),
    ),
)


def _tc_mid_body(part_ref, hws_ref, dis_ref, b_ref, w_ref, out_ref):
  dis = dis_ref[...]
  h = jnp.maximum(
      dis * (part_ref[0] + part_ref[1] + hws_ref[...]) + b_ref[...], 0.0)
  out_ref[...] = dis * _dot_t(h, w_ref[...])


_tc_mid = pl.pallas_call(
    _tc_mid_body,
    out_shape=jax.ShapeDtypeStruct((N, H), jnp.float32),
)


def _tc_final_body(part_ref, hws_ref, dis_ref, b_ref, batch_ref,
                   wc_ref, bc_ref, out_ref):
  h = jnp.maximum(
      dis_ref[...] * (part_ref[0] + part_ref[1] + hws_ref[...]) + b_ref[...],
      0.0)
  ids = lax.broadcasted_iota(jnp.int32, (B, N), 0)
  m = (batch_ref[...] == ids).astype(jnp.float32)
  cnt = jnp.sum(m, axis=1, keepdims=True)
  pooled = jnp.dot(m, h, preferred_element_type=jnp.float32)
  pooled = pooled / jnp.maximum(cnt, 1.0)
  out_ref[...] = _dot_t(pooled, wc_ref[...]) + bc_ref[...]


_tc_final = pl.pallas_call(
    _tc_final_body,
    out_shape=jax.ShapeDtypeStruct((B, C), jnp.float32),
)


@jax.jit
def kernel(x, edge_index, edge_attr, batch, W1, b1, W2, b2, W3, b3, Wc, bc):
  row = edge_index[0]
  col = edge_index[1]
  pad = E_PAD - E
  row_p = jnp.concatenate([row, jnp.zeros((pad,), jnp.int32)])
  col_p = jnp.concatenate([col, jnp.zeros((pad,), jnp.int32)])
  ea_p = jnp.concatenate([edge_attr, jnp.full((pad,), 1e30, jnp.float32)])
  colh_p = jnp.concatenate([col, jnp.full((pad,), N, jnp.int32)])
  zeros = jnp.zeros((ROWS_A, H), jnp.float32)
  zeros16 = jnp.zeros((ROWS_A, 16), jnp.float32)

  hist = _sc_hist(colh_p, zeros16)
  dis_col, hws1 = _tc_prep(hist, x, W1)

  part1, w = _edge_pass_first(hws1, row_p, col_p, ea_p, zeros)
  hws2 = _tc_mid(part1, hws1, dis_col, b1.reshape(1, H), W2)
  part2 = _edge_pass_rest(hws2, row_p, col_p, w, zeros)
  hws3 = _tc_mid(part2, hws2, dis_col, b2.reshape(1, H), W3)
  part3 = _edge_pass_rest(hws3, row_p, col_p, w, zeros)
  return _tc_final(part3, hws3, dis_col, b3.reshape(1, H),
                   batch.reshape(1, N), Wc, bc.reshape(1, C))


# speedup vs baseline: 15.9505x; 1.2825x over previous
import functools

import jax
import jax.numpy as jnp
from jax import lax
from jax.experimental import pallas as pl
from jax.experimental.pallas import tpu as pltpu
from jax.experimental.pallas import tpu_sc as plsc

N = 10000
E = 320000
D_IN = 128
H = 64
C = 100
B = 16

NC = 2
NS = 16
NW = NC * NS
CHUNK = 128
ROWS_A = 632
ROWS_B = N - (NS - 1) * ROWS_A

CHUNKS_PER_TILE = 80
E_PAD = NW * CHUNK * CHUNKS_PER_TILE
E_PER_TILE = E_PAD // NW
NCHUNKS = E_PAD // CHUNK
HIST_LAG = 8

_MESH = plsc.VectorSubcoreMesh(core_axis_name="c", subcore_axis_name="s",
                               num_cores=NC, num_subcores=NS)
_SC_PARAMS = pltpu.CompilerParams(use_tc_tiling_on_sc=False)


def _per_tile_copy(sid, make_src, make_dst):
  off = pl.multiple_of(sid * ROWS_A, 8)

  @pl.when(sid < NS - 1)
  def _():
    pltpu.sync_copy(make_src(off, ROWS_A), make_dst(off, ROWS_A))

  @pl.when(sid == NS - 1)
  def _():
    pltpu.sync_copy(make_src(off, ROWS_B), make_dst(off, ROWS_B))


@functools.partial(
    pl.kernel,
    out_type=jax.ShapeDtypeStruct((NC, N, 16), jnp.float32),
    mesh=_MESH,
    scratch_types=[
        pltpu.VMEM((CHUNKS_PER_TILE, CHUNK), jnp.int32),
        pltpu.VMEM((CHUNK, 16), jnp.float32),
        pltpu.VMEM_SHARED((N + 16, 16), jnp.float32),
        pltpu.SemaphoreType.DMA,
    ],
    compiler_params=_SC_PARAMS,
)
def _sc_hist(colh_hbm, zeros_hbm, out_hbm, col3_v, ones_v, acc_sh, ssem):
  cid = lax.axis_index("c")
  sid = lax.axis_index("s")
  wid = sid * NC + cid
  lanes = lax.iota(jnp.int32, 16)
  e0 = jnp.where(lanes == 0, 1.0, 0.0).astype(jnp.float32)
  for i in range(CHUNK):
    ones_v[i] = e0
  pltpu.sync_copy(colh_hbm.at[pl.ds(wid * CHUNKS_PER_TILE, CHUNKS_PER_TILE)],
                  col3_v)
  _per_tile_copy(sid, lambda o, r: zeros_hbm.at[pl.ds(0, r)],
                 lambda o, r: acc_sh.at[pl.ds(o, r)])

  @pl.when(sid == 0)
  def _():
    pltpu.sync_copy(zeros_hbm.at[pl.ds(0, 16)], acc_sh.at[pl.ds(N, 16)])

  plsc.subcore_barrier()

  def chunk(j, _):
    @pl.when(j >= HIST_LAG)
    def _():
      pltpu.make_async_copy(ones_v, acc_sh.at[col3_v.at[0]], ssem).wait()

    pltpu.async_copy(ones_v, acc_sh.at[col3_v.at[j]], ssem, add=True)
    return ()

  lax.fori_loop(0, CHUNKS_PER_TILE, chunk, ())
  for _ in range(HIST_LAG):
    pltpu.make_async_copy(ones_v, acc_sh.at[col3_v.at[0]], ssem).wait()
  plsc.subcore_barrier()
  _per_tile_copy(sid, lambda o, r: acc_sh.at[pl.ds(o, r)],
                 lambda o, r: out_hbm.at[cid, pl.ds(o, r)])


def _edge_pass_body(first, hw_hbm, row_hbm, col_hbm, w_hbm,
                    zeros_hbm, part_hbm, w_out_hbm,
                    row3_v, col3_v, w3_v, msg_bufs, acc_sh, gsem, ssem):
  cid = lax.axis_index("c")
  sid = lax.axis_index("s")
  wid = sid * NC + cid
  base_c = wid * CHUNKS_PER_TILE
  _per_tile_copy(sid, lambda o, r: zeros_hbm.at[pl.ds(0, r)],
                 lambda o, r: acc_sh.at[pl.ds(o, r)])
  pltpu.sync_copy(row_hbm.at[pl.ds(base_c, CHUNKS_PER_TILE)], row3_v)
  pltpu.sync_copy(col_hbm.at[pl.ds(base_c, CHUNKS_PER_TILE)], col3_v)
  pltpu.sync_copy(w_hbm.at[pl.ds(base_c, CHUNKS_PER_TILE)], w3_v)
  if first:
    def wrow(j, _):
      for g in range(CHUNK // 16):
        sl = pl.ds(g * 16, 16)
        w3_v[j, sl] = jnp.exp(-w3_v[j, sl])
      return ()

    lax.fori_loop(0, CHUNKS_PER_TILE, wrow, ())
    pltpu.sync_copy(w3_v, w_out_hbm.at[pl.ds(base_c, CHUNKS_PER_TILE)])
  plsc.subcore_barrier()

  def gather(j, buf):
    return pltpu.async_copy(hw_hbm.at[row3_v.at[j]], buf, gsem)

  def wait_scatter(buf):
    pltpu.make_async_copy(buf, acc_sh.at[col3_v.at[0]], ssem).wait()

  gather(0, msg_bufs[0])

  def outer(jo, _):
    for b in range(2):
      j = jo * 2 + b
      buf = msg_bufs[b]
      other = msg_bufs[1 - b]
      pltpu.make_async_copy(hw_hbm.at[row3_v.at[0]], buf, gsem).wait()

      @pl.when(jnp.logical_and(j >= 1, j < CHUNKS_PER_TILE - 1))
      def _():
        wait_scatter(other)

      @pl.when(j < CHUNKS_PER_TILE - 1)
      def _():
        gather(j + 1, other)

      def scale(g, _):
        wv = w3_v[j, pl.ds(g * 16, 16)]
        for l in range(16):
          s = jnp.full((16,), wv[l], jnp.float32)
          r_idx = g * 16 + l
          for r in range(H // 16):
            sl = pl.ds(r * 16, 16)
            buf[r_idx, sl] = buf[r_idx, sl] * s
        return ()

      lax.fori_loop(0, CHUNK // 16, scale, ())
      pltpu.async_copy(buf, acc_sh.at[col3_v.at[j]], ssem, add=True)
    return ()

  lax.fori_loop(0, CHUNKS_PER_TILE // 2, outer, ())
  wait_scatter(msg_bufs[0])
  wait_scatter(msg_bufs[1])
  plsc.subcore_barrier()
  _per_tile_copy(sid, lambda o, r: acc_sh.at[pl.ds(o, r)],
                 lambda o, r: part_hbm.at[cid, pl.ds(o, r)])


def _make_edge_pass(first):
  scratch = [
      pltpu.VMEM((CHUNKS_PER_TILE, CHUNK), jnp.int32),
      pltpu.VMEM((CHUNKS_PER_TILE, CHUNK), jnp.int32),
      pltpu.VMEM((CHUNKS_PER_TILE, CHUNK), jnp.float32),
      pltpu.VMEM((CHUNK, H), jnp.float32),
      pltpu.VMEM((CHUNK, H), jnp.float32),
      pltpu.VMEM_SHARED((N, H), jnp.float32),
      pltpu.SemaphoreType.DMA,
      pltpu.SemaphoreType.DMA,
  ]
  part_t = jax.ShapeDtypeStruct((NC, N, H), jnp.float32)

  if first:
    out_type = (part_t,
                jax.ShapeDtypeStruct((NCHUNKS, CHUNK), jnp.float32))

    def body(hw, row, col, ea, zeros, part, w_out, r3, c3, w3, m0, m1,
             acc, gsem, ssem):
      _edge_pass_body(True, hw, row, col, ea, zeros, part, w_out,
                      r3, c3, w3, (m0, m1), acc, gsem, ssem)
  else:
    out_type = part_t

    def body(hw, row, col, w, zeros, part, r3, c3, w3, m0, m1,
             acc, gsem, ssem):
      _edge_pass_body(False, hw, row, col, w, zeros, part, None,
                      r3, c3, w3, (m0, m1), acc, gsem, ssem)

  return pl.kernel(body, out_type=out_type, mesh=_MESH,
                   scratch_types=scratch, compiler_params=_SC_PARAMS)


_edge_pass_first = _make_edge_pass(True)
_edge_pass_rest = _make_edge_pass(False)


def _dot_t(a, w):
  return lax.dot_general(a, w, (((1,), (1,)), ((), ())),
                         preferred_element_type=jnp.float32)


def _tc_prep_body(hist_ref, x_ref, w1_ref, dis_ref, hws_ref):
  deg = hist_ref[0, :, 0:1] + hist_ref[1, :, 0:1] + 1.0
  dis = lax.rsqrt(deg)
  dis_ref[...] = dis
  hws_ref[...] = dis * _dot_t(x_ref[...], w1_ref[...])


_tc_prep = pl.pallas_call(
    _tc_prep_body,
    out_shape=(
        jax.ShapeDtypeStruct((N, 1), jnp.float32),
        jax.ShapeDtypeStruct((N, H), jnp.float32),
    ),
)


def _tc_mid_body(part_ref, hws_ref, dis_ref, b_ref, w_ref, out_ref):
  dis = dis_ref[...]
  h = jnp.maximum(
      dis * (part_ref[0] + part_ref[1] + hws_ref[...]) + b_ref[...], 0.0)
  out_ref[...] = dis * _dot_t(h, w_ref[...])


_tc_mid = pl.pallas_call(
    _tc_mid_body,
    out_shape=jax.ShapeDtypeStruct((N, H), jnp.float32),
)


def _tc_final_body(part_ref, hws_ref, dis_ref, b_ref, batch_ref,
                   wc_ref, bc_ref, out_ref):
  h = jnp.maximum(
      dis_ref[...] * (part_ref[0] + part_ref[1] + hws_ref[...]) + b_ref[...],
      0.0)
  ids = lax.broadcasted_iota(jnp.int32, (B, N), 0)
  m = (batch_ref[...] == ids).astype(jnp.float32)
  cnt = jnp.sum(m, axis=1, keepdims=True)
  pooled = jnp.dot(m, h, preferred_element_type=jnp.float32)
  pooled = pooled / jnp.maximum(cnt, 1.0)
  out_ref[...] = _dot_t(pooled, wc_ref[...]) + bc_ref[...]


_tc_final = pl.pallas_call(
    _tc_final_body,
    out_shape=jax.ShapeDtypeStruct((B, C), jnp.float32),
)


@jax.jit
def kernel(x, edge_index, edge_attr, batch, W1, b1, W2, b2, W3, b3, Wc, bc):
  row = edge_index[0]
  col = edge_index[1]
  pad = E_PAD - E
  spread = jnp.arange(pad, dtype=jnp.int32)
  row_p = jnp.concatenate([row, spread % N])
  col_p = jnp.concatenate([col, spread % N])
  ea_p = jnp.concatenate([edge_attr, jnp.full((pad,), 1e30, jnp.float32)])
  colh_p = jnp.concatenate([col, N + (spread % 16)])
  row_p = row_p.reshape(NCHUNKS, CHUNK)
  col_p = col_p.reshape(NCHUNKS, CHUNK)
  ea_p = ea_p.reshape(NCHUNKS, CHUNK)
  colh_p = colh_p.reshape(NCHUNKS, CHUNK)
  zeros = jnp.zeros((ROWS_A, H), jnp.float32)
  zeros16 = jnp.zeros((ROWS_A, 16), jnp.float32)

  hist = _sc_hist(colh_p, zeros16)
  dis_col, hws1 = _tc_prep(hist, x, W1)

  part1, w = _edge_pass_first(hws1, row_p, col_p, ea_p, zeros)
  hws2 = _tc_mid(part1, hws1, dis_col, b1.reshape(1, H), W2)
  part2 = _edge_pass_rest(hws2, row_p, col_p, w, zeros)
  hws3 = _tc_mid(part2, hws2, dis_col, b2.reshape(1, H), W3)
  part3 = _edge_pass_rest(hws3, row_p, col_p, w, zeros)
  return _tc_final(part3, hws3, dis_col, b3.reshape(1, H),
                   batch.reshape(1, N), Wc, bc.reshape(1, C))


# speedup vs baseline: 17.8412x; 1.1185x over previous
import functools

import jax
import jax.numpy as jnp
from jax import lax
from jax.experimental import pallas as pl
from jax.experimental.pallas import tpu as pltpu
from jax.experimental.pallas import tpu_sc as plsc

N = 10000
E = 320000
D_IN = 128
H = 64
C = 100
B = 16

NC = 2
NS = 16
NW = NC * NS
CHUNK = 128
ROWS_A = 632
ROWS_B = N - (NS - 1) * ROWS_A

CHUNKS_PER_TILE = 80
E_PAD = NW * CHUNK * CHUNKS_PER_TILE
E_PER_TILE = E_PAD // NW
NCHUNKS = E_PAD // CHUNK
HIST_LAG = 8

_MESH = plsc.VectorSubcoreMesh(core_axis_name="c", subcore_axis_name="s",
                               num_cores=NC, num_subcores=NS)
_SC_PARAMS = pltpu.CompilerParams(use_tc_tiling_on_sc=False)


def _per_tile_copy(sid, make_src, make_dst):
  off = pl.multiple_of(sid * ROWS_A, 8)

  @pl.when(sid < NS - 1)
  def _():
    pltpu.sync_copy(make_src(off, ROWS_A), make_dst(off, ROWS_A))

  @pl.when(sid == NS - 1)
  def _():
    pltpu.sync_copy(make_src(off, ROWS_B), make_dst(off, ROWS_B))


@functools.partial(
    pl.kernel,
    out_type=jax.ShapeDtypeStruct((NC, N, 16), jnp.float32),
    mesh=_MESH,
    scratch_types=[
        pltpu.VMEM((CHUNKS_PER_TILE, CHUNK), jnp.int32),
        pltpu.VMEM((CHUNK, 16), jnp.float32),
        pltpu.VMEM_SHARED((N + 16, 16), jnp.float32),
        pltpu.SemaphoreType.DMA,
    ],
    compiler_params=_SC_PARAMS,
)
def _sc_hist(colh_hbm, zeros_hbm, out_hbm, col3_v, ones_v, acc_sh, ssem):
  cid = lax.axis_index("c")
  sid = lax.axis_index("s")
  wid = sid * NC + cid
  lanes = lax.iota(jnp.int32, 16)
  e0 = jnp.where(lanes == 0, 1.0, 0.0).astype(jnp.float32)
  for i in range(CHUNK):
    ones_v[i] = e0
  pltpu.sync_copy(colh_hbm.at[pl.ds(wid * CHUNKS_PER_TILE, CHUNKS_PER_TILE)],
                  col3_v)
  _per_tile_copy(sid, lambda o, r: zeros_hbm.at[pl.ds(0, r)],
                 lambda o, r: acc_sh.at[pl.ds(o, r)])

  @pl.when(sid == 0)
  def _():
    pltpu.sync_copy(zeros_hbm.at[pl.ds(0, 16)], acc_sh.at[pl.ds(N, 16)])

  plsc.subcore_barrier()

  def chunk(j, _):
    @pl.when(j >= HIST_LAG)
    def _():
      pltpu.make_async_copy(ones_v, acc_sh.at[col3_v.at[0]], ssem).wait()

    pltpu.async_copy(ones_v, acc_sh.at[col3_v.at[j]], ssem, add=True)
    return ()

  lax.fori_loop(0, CHUNKS_PER_TILE, chunk, ())
  for _ in range(HIST_LAG):
    pltpu.make_async_copy(ones_v, acc_sh.at[col3_v.at[0]], ssem).wait()
  plsc.subcore_barrier()
  _per_tile_copy(sid, lambda o, r: acc_sh.at[pl.ds(o, r)],
                 lambda o, r: out_hbm.at[cid, pl.ds(o, r)])


def _edge_pass_body(first, hw_hbm, row_hbm, col_hbm, w_hbm,
                    zeros_hbm, part_hbm, w_out_hbm,
                    row3_v, col3_v, w3_v, msg_bufs, acc_sh, gsem, ssem):
  cid = lax.axis_index("c")
  sid = lax.axis_index("s")
  wid = sid * NC + cid
  base_c = wid * CHUNKS_PER_TILE
  _per_tile_copy(sid, lambda o, r: zeros_hbm.at[pl.ds(0, r)],
                 lambda o, r: acc_sh.at[pl.ds(o, r)])
  pltpu.sync_copy(row_hbm.at[pl.ds(base_c, CHUNKS_PER_TILE)], row3_v)
  pltpu.sync_copy(col_hbm.at[pl.ds(base_c, CHUNKS_PER_TILE)], col3_v)
  pltpu.sync_copy(w_hbm.at[pl.ds(base_c, CHUNKS_PER_TILE)], w3_v)
  if first:
    def wrow(j, _):
      for g in range(CHUNK // 16):
        sl = pl.ds(g * 16, 16)
        w3_v[j, sl] = jnp.exp(-w3_v[j, sl])
      return ()

    lax.fori_loop(0, CHUNKS_PER_TILE, wrow, ())
    pltpu.sync_copy(w3_v, w_out_hbm.at[pl.ds(base_c, CHUNKS_PER_TILE)])
  plsc.subcore_barrier()

  nbuf = len(msg_bufs)

  def gather(j, buf):
    return pltpu.async_copy(hw_hbm.at[row3_v.at[j]], buf, gsem)

  def wait_scatter(buf):
    pltpu.make_async_copy(buf, acc_sh.at[col3_v.at[0]], ssem).wait()

  gather(0, msg_bufs[0])

  def outer(jo, _):
    for b in range(nbuf):
      j = jo * nbuf + b
      buf = msg_bufs[b]
      nxt = msg_bufs[(b + 1) % nbuf]
      pltpu.make_async_copy(hw_hbm.at[row3_v.at[0]], buf, gsem).wait()

      @pl.when(jnp.logical_and(j + 1 >= nbuf, j < CHUNKS_PER_TILE - 1))
      def _():
        wait_scatter(nxt)

      @pl.when(j < CHUNKS_PER_TILE - 1)
      def _():
        gather(j + 1, nxt)

      def scale(g, _):
        wv = w3_v[j, pl.ds(g * 16, 16)]
        for l in range(16):
          s = jnp.full((16,), wv[l], jnp.float32)
          r_idx = g * 16 + l
          for r in range(H // 16):
            sl = pl.ds(r * 16, 16)
            buf[r_idx, sl] = buf[r_idx, sl] * s
        return ()

      lax.fori_loop(0, CHUNK // 16, scale, ())
      pltpu.async_copy(buf, acc_sh.at[col3_v.at[j]], ssem, add=True)
    return ()

  lax.fori_loop(0, CHUNKS_PER_TILE // nbuf, outer, ())
  for buf in msg_bufs:
    wait_scatter(buf)
  plsc.subcore_barrier()
  _per_tile_copy(sid, lambda o, r: acc_sh.at[pl.ds(o, r)],
                 lambda o, r: part_hbm.at[cid, pl.ds(o, r)])


def _make_edge_pass(first):
  scratch = [
      pltpu.VMEM((CHUNKS_PER_TILE, CHUNK), jnp.int32),
      pltpu.VMEM((CHUNKS_PER_TILE, CHUNK), jnp.int32),
      pltpu.VMEM((CHUNKS_PER_TILE, CHUNK), jnp.float32),
      pltpu.VMEM((CHUNK, H), jnp.float32),
      pltpu.VMEM((CHUNK, H), jnp.float32),
      pltpu.VMEM((CHUNK, H), jnp.float32),
      pltpu.VMEM((CHUNK, H), jnp.float32),
      pltpu.VMEM_SHARED((N, H), jnp.float32),
      pltpu.SemaphoreType.DMA,
      pltpu.SemaphoreType.DMA,
  ]
  part_t = jax.ShapeDtypeStruct((NC, N, H), jnp.float32)

  if first:
    out_type = (part_t,
                jax.ShapeDtypeStruct((NCHUNKS, CHUNK), jnp.float32))

    def body(hw, row, col, ea, zeros, part, w_out, r3, c3, w3,
             m0, m1, m2, m3, acc, gsem, ssem):
      _edge_pass_body(True, hw, row, col, ea, zeros, part, w_out,
                      r3, c3, w3, (m0, m1, m2, m3), acc, gsem, ssem)
  else:
    out_type = part_t

    def body(hw, row, col, w, zeros, part, r3, c3, w3,
             m0, m1, m2, m3, acc, gsem, ssem):
      _edge_pass_body(False, hw, row, col, w, zeros, part, None,
                      r3, c3, w3, (m0, m1, m2, m3), acc, gsem, ssem)

  return pl.kernel(body, out_type=out_type, mesh=_MESH,
                   scratch_types=scratch, compiler_params=_SC_PARAMS)


_edge_pass_first = _make_edge_pass(True)
_edge_pass_rest = _make_edge_pass(False)


def _dot_t(a, w):
  return lax.dot_general(a, w, (((1,), (1,)), ((), ())),
                         preferred_element_type=jnp.float32)


def _tc_prep_body(hist_ref, x_ref, w1_ref, dis_ref, hws_ref):
  deg = hist_ref[0, :, 0:1] + hist_ref[1, :, 0:1] + 1.0
  dis = lax.rsqrt(deg)
  dis_ref[...] = dis
  hws_ref[...] = dis * _dot_t(x_ref[...], w1_ref[...])


_tc_prep = pl.pallas_call(
    _tc_prep_body,
    out_shape=(
        jax.ShapeDtypeStruct((N, 1), jnp.float32),
        jax.ShapeDtypeStruct((N, H), jnp.float32),
    ),
)


def _tc_mid_body(part_ref, hws_ref, dis_ref, b_ref, w_ref, out_ref):
  dis = dis_ref[...]
  h = jnp.maximum(
      dis * (part_ref[0] + part_ref[1] + hws_ref[...]) + b_ref[...], 0.0)
  out_ref[...] = dis * _dot_t(h, w_ref[...])


_tc_mid = pl.pallas_call(
    _tc_mid_body,
    out_shape=jax.ShapeDtypeStruct((N, H), jnp.float32),
)


def _tc_final_body(part_ref, hws_ref, dis_ref, b_ref, batch_ref,
                   wc_ref, bc_ref, out_ref):
  h = jnp.maximum(
      dis_ref[...] * (part_ref[0] + part_ref[1] + hws_ref[...]) + b_ref[...],
      0.0)
  ids = lax.broadcasted_iota(jnp.int32, (B, N), 0)
  m = (batch_ref[...] == ids).astype(jnp.float32)
  cnt = jnp.sum(m, axis=1, keepdims=True)
  pooled = jnp.dot(m, h, preferred_element_type=jnp.float32)
  pooled = pooled / jnp.maximum(cnt, 1.0)
  out_ref[...] = _dot_t(pooled, wc_ref[...]) + bc_ref[...]


_tc_final = pl.pallas_call(
    _tc_final_body,
    out_shape=jax.ShapeDtypeStruct((B, C), jnp.float32),
)


@jax.jit
def kernel(x, edge_index, edge_attr, batch, W1, b1, W2, b2, W3, b3, Wc, bc):
  row = edge_index[0]
  col = edge_index[1]
  pad = E_PAD - E
  spread = jnp.arange(pad, dtype=jnp.int32)
  row_p = jnp.concatenate([row, spread % N])
  col_p = jnp.concatenate([col, spread % N])
  ea_p = jnp.concatenate([edge_attr, jnp.full((pad,), 1e30, jnp.float32)])
  colh_p = jnp.concatenate([col, N + (spread % 16)])
  row_p = row_p.reshape(NCHUNKS, CHUNK)
  col_p = col_p.reshape(NCHUNKS, CHUNK)
  ea_p = ea_p.reshape(NCHUNKS, CHUNK)
  colh_p = colh_p.reshape(NCHUNKS, CHUNK)
  zeros = jnp.zeros((ROWS_A, H), jnp.float32)
  zeros16 = jnp.zeros((ROWS_A, 16), jnp.float32)

  hist = _sc_hist(colh_p, zeros16)
  dis_col, hws1 = _tc_prep(hist, x, W1)

  part1, w = _edge_pass_first(hws1, row_p, col_p, ea_p, zeros)
  hws2 = _tc_mid(part1, hws1, dis_col, b1.reshape(1, H), W2)
  part2 = _edge_pass_rest(hws2, row_p, col_p, w, zeros)
  hws3 = _tc_mid(part2, hws2, dis_col, b2.reshape(1, H), W3)
  part3 = _edge_pass_rest(hws3, row_p, col_p, w, zeros)
  return _tc_final(part3, hws3, dis_col, b3.reshape(1, H),
                   batch.reshape(1, N), Wc, bc.reshape(1, C))
